# baseline (speedup 1.0000x reference)
import functools
import math

import jax
import jax.numpy as jnp
import numpy as np
from jax import lax
from jax.experimental import pallas as pl
from jax.experimental.pallas import tpu as pltpu
from jax.experimental.pallas import tpu_sc as plsc

TOKEN_DIM = 64
L_SEQ = 200
B_SEQ = 1024
NFLAT = B_SEQ * L_SEQ
NW = 32
PER_W = NFLAT // NW
GSZ = 128
NG = PER_W // GSZ
VREGS = TOKEN_DIM // 16


def _build_pe() -> np.ndarray:
    pe = np.zeros((L_SEQ, TOKEN_DIM), dtype=np.float32)
    positions = np.arange(0, L_SEQ, dtype=np.float32)[:, None]
    div_term = np.exp(
        np.arange(0, TOKEN_DIM, 2, dtype=np.float32) * -(math.log(10000.0) / TOKEN_DIM)
    )
    pe[:, 0::2] = np.sin(positions * div_term)
    pe[:, 1::2] = np.cos(positions * div_term)
    return pe


_MESH = plsc.VectorSubcoreMesh(core_axis_name="c", subcore_axis_name="s")


@functools.partial(
    pl.kernel,
    mesh=_MESH,
    compiler_params=pltpu.CompilerParams(use_tc_tiling_on_sc=False),
    out_type=jax.ShapeDtypeStruct((NFLAT, TOKEN_DIM), jnp.float32),
    scratch_types=[
        pltpu.VMEM((NG, GSZ), jnp.int32),
        pltpu.VMEM((GSZ, TOKEN_DIM), jnp.float32),
        pltpu.VMEM((L_SEQ, TOKEN_DIM), jnp.float32),
        pltpu.SemaphoreType.DMA,
    ],
)
def _emb_kernel(table_hbm, idx_hbm, pe_hbm, out_hbm, idx_v, rows_v, pe_v, sem):
    wid = lax.axis_index("s") * 2 + lax.axis_index("c")
    base = wid * PER_W
    pltpu.sync_copy(pe_hbm, pe_v)
    pltpu.sync_copy(idx_hbm.at[wid], idx_v)

    def gbody(g, carry):
        pltpu.async_copy(table_hbm.at[idx_v.at[g]], rows_v, sem).wait()
        gb = base + g * GSZ

        def jbody(j, c2):
            l = lax.rem(gb + j, L_SEQ)
            for k in range(VREGS):
                sl = pl.ds(k * 16, 16)
                rows_v[j, sl] = rows_v[j, sl] * 8.0 + pe_v[l, sl]
            return c2

        lax.fori_loop(0, GSZ, jbody, 0)
        pltpu.sync_copy(rows_v, out_hbm.at[pl.ds(gb, GSZ)])
        return carry

    lax.fori_loop(0, NG, gbody, 0)


def kernel(tgt, emb_table):
    idx2 = tgt.reshape(NW, NG, GSZ)
    pe = jnp.asarray(_build_pe())
    out = _emb_kernel(emb_table, idx2, pe)
    return out.reshape(B_SEQ, L_SEQ, TOKEN_DIM)
